# baseline (speedup 1.0000x reference)
import functools

import jax
import jax.numpy as jnp
from jax import lax
from jax.experimental import pallas as pl
from jax.experimental.pallas import tpu as pltpu
from jax.experimental.pallas import tpu_sc as plsc

STREAM = 128
UNIT = 256
IPT = 512


@functools.lru_cache(maxsize=None)
def _make(W, N, D):
    info = plsc.get_sparse_core_info()
    NC, NS = info.num_cores, info.num_subcores
    NW = NC * NS
    assert N == NW * IPT and IPT % UNIT == 0 and UNIT % STREAM == 0
    upw = IPT // UNIT
    n_units = W * upw
    mesh = plsc.VectorSubcoreMesh(core_axis_name="c", subcore_axis_name="s")

    @functools.partial(
        pl.kernel,
        mesh=mesh,
        compiler_params=pltpu.CompilerParams(
            needs_layout_passes=False, use_tc_tiling_on_sc=True),
        out_type=jax.ShapeDtypeStruct((W, D, N), jnp.float32),
        scratch_types=[
            pltpu.VMEM((W, IPT), jnp.int32),
            pltpu.VMEM((W, IPT), jnp.int32),
            pltpu.VMEM((UNIT, 2 * D), jnp.float32),
            pltpu.VMEM((UNIT, 2 * D), jnp.float32),
            pltpu.VMEM((D, UNIT), jnp.float32),
            pltpu.VMEM((D, UNIT), jnp.float32),
            pltpu.SemaphoreType.DMA,
            pltpu.SemaphoreType.DMA,
            pltpu.SemaphoreType.DMA,
            pltpu.SemaphoreType.DMA,
        ],
    )
    def k(nodesT_hbm, tpairs_hbm, out_hbm, idx_v, pidx_v, rb0, rb1, tb0, tb1,
          gsem0, gsem1, wsem0, wsem1):
        wid = lax.axis_index("s") * NC + lax.axis_index("c")
        i0 = wid * IPT
        pltpu.sync_copy(nodesT_hbm.at[:, pl.ds(i0, IPT)], idx_v)
        lanes = lax.iota(jnp.int32, 16)

        def halve(kk, carry):
            for w in range(W):
                v = idx_v[w, pl.ds(kk * 16, 16)]
                pidx_v[w, pl.ds(kk * 16, 16)] = v >> 1
            return carry

        lax.fori_loop(0, IPT // 16, halve, 0)

        def fire(u, rb, gsem):
            w = u // upw
            col = (u % upw) * UNIT
            return [
                pltpu.async_copy(
                    tpairs_hbm.at[pidx_v.at[w, pl.ds(col + s * STREAM, STREAM)]],
                    rb.at[pl.ds(s * STREAM, STREAM)], gsem)
                for s in range(UNIT // STREAM)
            ]

        def shuffle(u, rb, tb):
            w = u // upw
            col = (u % upw) * UNIT

            def body(j, carry):
                rows = j * 16 + lanes
                iv = idx_v[w, pl.ds(col + j * 16, 16)]
                par = (iv & 1) * D
                for d in range(D):
                    vals = plsc.load_gather(rb, [rows, par + d])
                    tb[d, pl.ds(j * 16, 16)] = vals
                return carry

            lax.fori_loop(0, UNIT // 16, body, 0)

        def wstart(u, tb, wsem):
            w = u // upw
            col = i0 + (u % upw) * UNIT
            return pltpu.async_copy(tb, out_hbm.at[w, :, pl.ds(col, UNIT)], wsem)

        def body(t, carry):
            e = 2 * t
            o = e + 1
            ge = fire(e, rb0, gsem0)
            go = fire(o, rb1, gsem1)
            for h in ge:
                h.wait()
            shuffle(e, rb0, tb0)
            we = wstart(e, tb0, wsem0)
            for h in go:
                h.wait()
            shuffle(o, rb1, tb1)
            wo = wstart(o, tb1, wsem1)
            we.wait()
            wo.wait()
            return carry

        lax.fori_loop(0, n_units // 2, body, 0)

    return k


def kernel(nodes, table):
    n, w = nodes.shape
    v, d = table.shape
    nodesT = nodes.T.astype(jnp.int32)
    tpairs = table.reshape(v // 2, 2 * d)
    outT = _make(w, n, d)(nodesT, tpairs)
    return outT.transpose(2, 0, 1)


# speedup vs baseline: 1.3005x; 1.3005x over previous
import functools

import jax
import jax.numpy as jnp
from jax import lax
from jax.experimental import pallas as pl
from jax.experimental.pallas import tpu as pltpu
from jax.experimental.pallas import tpu_sc as plsc

STREAM = 128
UNIT = 256
IPT = 512


@functools.lru_cache(maxsize=None)
def _make(W, N, D):
    info = plsc.get_sparse_core_info()
    NC, NS = info.num_cores, info.num_subcores
    NW = NC * NS
    assert N == NW * IPT and IPT % UNIT == 0 and UNIT % STREAM == 0
    upw = IPT // UNIT
    n_units = W * upw
    mesh = plsc.VectorSubcoreMesh(core_axis_name="c", subcore_axis_name="s")

    @functools.partial(
        pl.kernel,
        mesh=mesh,
        compiler_params=pltpu.CompilerParams(
            needs_layout_passes=False, use_tc_tiling_on_sc=True),
        out_type=jax.ShapeDtypeStruct((W, D, N), jnp.float32),
        scratch_types=[
            pltpu.VMEM((W, IPT), jnp.int32),
            pltpu.VMEM((UNIT, 2 * D), jnp.float32),
            pltpu.VMEM((UNIT, 2 * D), jnp.float32),
            pltpu.VMEM((D, UNIT), jnp.float32),
            pltpu.VMEM((D, UNIT), jnp.float32),
            pltpu.SemaphoreType.DMA,
            pltpu.SemaphoreType.DMA,
            pltpu.SemaphoreType.DMA,
            pltpu.SemaphoreType.DMA,
        ],
    )
    def k(nodesT_hbm, tpad_hbm, out_hbm, idx_v, rb0, rb1, tb0, tb1,
          gsem0, gsem1, wsem0, wsem1):
        wid = lax.axis_index("s") * NC + lax.axis_index("c")
        i0 = wid * IPT
        pltpu.sync_copy(nodesT_hbm.at[:, pl.ds(i0, IPT)], idx_v)
        lanes = lax.iota(jnp.int32, 16)

        def fire(u, rb, gsem):
            w = u // upw
            col = (u % upw) * UNIT
            return [
                pltpu.async_copy(
                    tpad_hbm.at[idx_v.at[w, pl.ds(col + s * STREAM, STREAM)]],
                    rb.at[pl.ds(s * STREAM, STREAM)], gsem)
                for s in range(UNIT // STREAM)
            ]

        def shuffle(rb, tb):
            @plsc.parallel_loop(0, UNIT // 16)
            def body(j):
                rows = j * 16 + lanes
                for d in range(D):
                    vals = plsc.load_gather(rb, [rows, jnp.full((16,), d, jnp.int32)])
                    tb[d, pl.ds(j * 16, 16)] = vals

        def wstart(u, tb, wsem):
            w = u // upw
            col = i0 + (u % upw) * UNIT
            return pltpu.async_copy(tb, out_hbm.at[w, :, pl.ds(col, UNIT)], wsem)

        def body(t, carry):
            e = 2 * t
            o = e + 1
            ge = fire(e, rb0, gsem0)
            go = fire(o, rb1, gsem1)
            for h in ge:
                h.wait()
            shuffle(rb0, tb0)
            we = wstart(e, tb0, wsem0)
            for h in go:
                h.wait()
            shuffle(rb1, tb1)
            wo = wstart(o, tb1, wsem1)
            we.wait()
            wo.wait()
            return carry

        lax.fori_loop(0, n_units // 2, body, 0)

    return k


def kernel(nodes, table):
    n, w = nodes.shape
    v, d = table.shape
    nodesT = nodes.T.astype(jnp.int32)
    tpad = jnp.pad(table, ((0, 0), (0, d)))
    outT = _make(w, n, d)(nodesT, tpad)
    return outT.transpose(2, 0, 1)


# speedup vs baseline: 1.4266x; 1.0970x over previous
import functools

import jax
import jax.numpy as jnp
from jax import lax
from jax.experimental import pallas as pl
from jax.experimental.pallas import tpu as pltpu
from jax.experimental.pallas import tpu_sc as plsc

STREAM = 128
UNIT = 256
IPT = 512


@functools.lru_cache(maxsize=None)
def _make(W, N, D):
    info = plsc.get_sparse_core_info()
    NC, NS = info.num_cores, info.num_subcores
    NW = NC * NS
    assert N == NW * IPT and IPT % UNIT == 0 and UNIT % STREAM == 0
    upw = IPT // UNIT
    n_units = W * upw
    mesh = plsc.VectorSubcoreMesh(core_axis_name="c", subcore_axis_name="s")

    @functools.partial(
        pl.kernel,
        mesh=mesh,
        compiler_params=pltpu.CompilerParams(
            needs_layout_passes=False, use_tc_tiling_on_sc=True),
        out_type=jax.ShapeDtypeStruct((W, D, N), jnp.float32),
        scratch_types=[
            pltpu.VMEM((W, IPT), jnp.int32),
            pltpu.VMEM((UNIT, 2 * D), jnp.float32),
            pltpu.VMEM((UNIT, 2 * D), jnp.float32),
            pltpu.VMEM((D, UNIT), jnp.float32),
            pltpu.VMEM((D, UNIT), jnp.float32),
            pltpu.SemaphoreType.DMA,
            pltpu.SemaphoreType.DMA,
            pltpu.SemaphoreType.DMA,
            pltpu.SemaphoreType.DMA,
        ],
    )
    def k(nodesT_hbm, tpad_hbm, out_hbm, idx_v, rb0, rb1, tb0, tb1,
          gsem0, gsem1, wsem0, wsem1):
        wid = lax.axis_index("s") * NC + lax.axis_index("c")
        i0 = wid * IPT
        pltpu.sync_copy(nodesT_hbm.at[:, pl.ds(i0, IPT)], idx_v)
        lanes = lax.iota(jnp.int32, 16)

        def fire(u, rb, gsem):
            w = u // upw
            col = (u % upw) * UNIT
            return [
                pltpu.async_copy(
                    tpad_hbm.at[idx_v.at[w, pl.ds(col + s * STREAM, STREAM)]],
                    rb.at[pl.ds(s * STREAM, STREAM)], gsem)
                for s in range(UNIT // STREAM)
            ]

        def shuffle(rb, tb):
            @plsc.parallel_loop(0, UNIT // 16)
            def body(j):
                rows = j * 16 + lanes
                for d in range(D):
                    vals = plsc.load_gather(rb, [rows, jnp.full((16,), d, jnp.int32)])
                    tb[d, pl.ds(j * 16, 16)] = vals

        def wstart(u, tb, wsem):
            w = u // upw
            col = i0 + (u % upw) * UNIT
            return pltpu.async_copy(tb, out_hbm.at[w, :, pl.ds(col, UNIT)], wsem)

        def gwait(rb, gsem):
            for s in range(UNIT // STREAM):
                pltpu.make_async_copy(
                    tpad_hbm.at[idx_v.at[0, pl.ds(s * STREAM, STREAM)]],
                    rb.at[pl.ds(s * STREAM, STREAM)], gsem).wait()

        def wwait(tb, wsem):
            pltpu.make_async_copy(tb, out_hbm.at[0, :, pl.ds(i0, UNIT)],
                                  wsem).wait()

        fire(0, rb0, gsem0)
        fire(1, rb1, gsem1)

        def body(t, carry):
            e = 2 * t
            o = e + 1

            @pl.when(t > 0)
            def _():
                wwait(tb0, wsem0)
            gwait(rb0, gsem0)
            shuffle(rb0, tb0)

            @pl.when(t < n_units // 2 - 1)
            def _():
                fire(e + 2, rb0, gsem0)
            wstart(e, tb0, wsem0)

            @pl.when(t > 0)
            def _():
                wwait(tb1, wsem1)
            gwait(rb1, gsem1)
            shuffle(rb1, tb1)

            @pl.when(t < n_units // 2 - 1)
            def _():
                fire(o + 2, rb1, gsem1)
            wstart(o, tb1, wsem1)
            return carry

        lax.fori_loop(0, n_units // 2, body, 0)
        wwait(tb0, wsem0)
        wwait(tb1, wsem1)

    return k


def kernel(nodes, table):
    n, w = nodes.shape
    v, d = table.shape
    nodesT = nodes.T.astype(jnp.int32)
    tpad = jnp.pad(table, ((0, 0), (0, d)))
    outT = _make(w, n, d)(nodesT, tpad)
    return outT.transpose(2, 0, 1)
